# Initial kernel scaffold; baseline (speedup 1.0000x reference)
import jax
import jax.numpy as jnp
from jax.experimental import pallas as pl


def kernel(input_ids, table):
    raise NotImplementedError("write your pallas kernel here")


# speedup vs baseline: 4.8071x; 4.8071x over previous
import functools

import jax
import jax.numpy as jnp
from jax import lax
from jax.experimental import pallas as pl
from jax.experimental.pallas import tpu as pltpu
from jax.experimental.pallas import tpu_sc as plsc

NUM_EMB = 1000000
D = 32
B_TOTAL = 16384 * 200

NC, NS = 2, 16
NW = NC * NS
GATHER = 128
G_PER_CHUNK = 8
CHUNK = GATHER * G_PER_CHUNK
ROWS_PER_W = B_TOTAL // NW
N_CHUNK = ROWS_PER_W // CHUNK
IDX_ROWS = B_TOTAL // GATHER

_mesh = plsc.VectorSubcoreMesh(core_axis_name="c", subcore_axis_name="s")


@functools.partial(
    pl.kernel,
    mesh=_mesh,
    out_type=jax.ShapeDtypeStruct((IDX_ROWS, GATHER, D), jnp.float32),
    scratch_types=[
        pltpu.VMEM((G_PER_CHUNK, GATHER), jnp.int32),
        pltpu.VMEM((G_PER_CHUNK, GATHER, D), jnp.float32),
        pltpu.SemaphoreType.DMA,
    ],
    compiler_params=pltpu.CompilerParams(use_tc_tiling_on_sc=False),
)
def _emb_lookup(idx_hbm, table_hbm, out_hbm, idx_v, rows_v, sem_g):
    wid = lax.axis_index("s") * NC + lax.axis_index("c")
    row0 = wid * (N_CHUNK * G_PER_CHUNK)

    def chunk_body(c, carry):
        irow = row0 + c * G_PER_CHUNK
        pltpu.sync_copy(idx_hbm.at[pl.ds(irow, G_PER_CHUNK)], idx_v)
        copies = [
            pltpu.async_copy(table_hbm.at[idx_v.at[j]], rows_v.at[j], sem_g)
            for j in range(G_PER_CHUNK)
        ]
        for cp in copies:
            cp.wait()
        pltpu.sync_copy(rows_v, out_hbm.at[pl.ds(irow, G_PER_CHUNK)])
        return carry

    lax.fori_loop(0, N_CHUNK, chunk_body, 0)


def kernel(input_ids, table):
    idx = input_ids.reshape(IDX_ROWS, GATHER).astype(jnp.int32)
    out = _emb_lookup(idx, table)
    return out.reshape(input_ids.shape[0], input_ids.shape[1], D)


# speedup vs baseline: 5.0323x; 1.0469x over previous
import functools

import jax
import jax.numpy as jnp
from jax import lax
from jax.experimental import pallas as pl
from jax.experimental.pallas import tpu as pltpu
from jax.experimental.pallas import tpu_sc as plsc

NUM_EMB = 1000000
D = 32
B_TOTAL = 16384 * 200

NC, NS = 2, 16
NW = NC * NS
GATHER = 128
G_PER_CHUNK = 8
CHUNK = GATHER * G_PER_CHUNK
ROWS_PER_W = B_TOTAL // NW
N_CHUNK = ROWS_PER_W // CHUNK
IDX_ROWS = B_TOTAL // GATHER

_mesh = plsc.VectorSubcoreMesh(core_axis_name="c", subcore_axis_name="s")


@functools.partial(
    pl.kernel,
    mesh=_mesh,
    out_type=jax.ShapeDtypeStruct((IDX_ROWS, GATHER, D), jnp.float32),
    scratch_types=[
        pltpu.VMEM((2, G_PER_CHUNK, GATHER), jnp.int32),
        pltpu.VMEM((2, G_PER_CHUNK, GATHER, D), jnp.float32),
        pltpu.SemaphoreType.DMA,
        pltpu.SemaphoreType.DMA,
        pltpu.SemaphoreType.DMA,
        pltpu.SemaphoreType.DMA,
    ],
    compiler_params=pltpu.CompilerParams(use_tc_tiling_on_sc=False),
)
def _emb_lookup(idx_hbm, table_hbm, out_hbm, idx_v, rows_v,
                sem_idx, sem_g, sem_out0, sem_out1):
    wid = lax.axis_index("s") * NC + lax.axis_index("c")
    row0 = wid * (N_CHUNK * G_PER_CHUNK)
    sem_out = (sem_out0, sem_out1)

    pltpu.async_copy(idx_hbm.at[pl.ds(row0, G_PER_CHUNK)], idx_v.at[0], sem_idx)

    def pair_body(g, carry):
        for b in range(2):
            c = 2 * g + b
            irow = row0 + c * G_PER_CHUNK

            @pl.when(g > 0)
            def _():
                pltpu.make_async_copy(
                    rows_v.at[b], out_hbm.at[pl.ds(irow, G_PER_CHUNK)],
                    sem_out[b]).wait()

            pltpu.make_async_copy(
                idx_hbm.at[pl.ds(irow, G_PER_CHUNK)], idx_v.at[b],
                sem_idx).wait()

            copies = [
                pltpu.async_copy(table_hbm.at[idx_v.at[b].at[j]],
                                 rows_v.at[b].at[j], sem_g)
                for j in range(G_PER_CHUNK)
            ]

            nrow = row0 + jnp.minimum(c + 1, N_CHUNK - 1) * G_PER_CHUNK
            pltpu.async_copy(idx_hbm.at[pl.ds(nrow, G_PER_CHUNK)],
                             idx_v.at[1 - b], sem_idx)

            for cp in copies:
                cp.wait()
            pltpu.async_copy(rows_v.at[b], out_hbm.at[pl.ds(irow, G_PER_CHUNK)],
                             sem_out[b])
        return carry

    lax.fori_loop(0, N_CHUNK // 2, pair_body, 0)

    pltpu.make_async_copy(idx_hbm.at[pl.ds(row0, G_PER_CHUNK)], idx_v.at[1],
                          sem_idx).wait()
    for b in range(2):
        tail = row0 + (N_CHUNK - 2 + b) * G_PER_CHUNK
        pltpu.make_async_copy(rows_v.at[b], out_hbm.at[pl.ds(tail, G_PER_CHUNK)],
                              sem_out[b]).wait()


def kernel(input_ids, table):
    idx = input_ids.reshape(IDX_ROWS, GATHER).astype(jnp.int32)
    out = _emb_lookup(idx, table)
    return out.reshape(input_ids.shape[0], input_ids.shape[1], D)
